# baseline (speedup 1.0000x reference)
import functools

import jax
import jax.numpy as jnp
from jax import lax
from jax.experimental import pallas as pl
from jax.experimental.pallas import tpu as pltpu
from jax.experimental.pallas import tpu_sc as plsc

_GRP = 128
_L = 16


def _gather_body(
    b_per_w, tab2_hbm, ids_hbm, out_hbm, ids_v, q_v, hb_v, rows_v, stage_v, sems
):
    wid = lax.axis_index("s") * 2 + lax.axis_index("c")
    base = wid * b_per_w
    pltpu.sync_copy(ids_hbm.at[pl.ds(base, b_per_w)], ids_v)

    vec_per_grp = _GRP // _L
    for i in range(b_per_w // _L):
        v = ids_v[pl.ds(i * _L, _L)]
        g, o = i // vec_per_grp, (i % vec_per_grp) * _L
        q_v[g, pl.ds(o, _L)] = v >> 1
        hb_v[g, pl.ds(o, _L)] = (v & 1) * 64

    for g in range(b_per_w // _GRP):
        pltpu.async_copy(tab2_hbm.at[q_v.at[g]], rows_v, sems.at[0]).wait()

        def per_batch(b, _):
            j0 = b * _L
            hv = hb_v[g, pl.ds(j0, _L)]
            for i in range(_L):
                j = j0 + i
                h = hv[i]
                for k in range(4):
                    stage_v[j, pl.ds(k * _L, _L)] = rows_v[
                        j, pl.ds(h + k * _L, _L)
                    ]
            return _

        lax.fori_loop(0, _GRP // _L, per_batch, 0, unroll=False)
        pltpu.sync_copy(
            stage_v, out_hbm.at[pl.ds(base + g * _GRP, _GRP)]
        )


@jax.jit
def _embed_lookup(node_ids, node_embed_weight):
    b = node_ids.shape[0]
    nrows = node_embed_weight.shape[0]
    tab2 = node_embed_weight.reshape(nrows // 2, 128)
    info = plsc.get_sparse_core_info()
    nw = info.num_cores * info.num_subcores
    b_per_w = b // nw
    n_grp = b_per_w // _GRP
    mesh = plsc.VectorSubcoreMesh(core_axis_name="c", subcore_axis_name="s")
    k = pl.kernel(
        functools.partial(_gather_body, b_per_w),
        mesh=mesh,
        out_type=jax.ShapeDtypeStruct((b, 64), jnp.float32),
        scratch_types=[
            pltpu.VMEM((b_per_w,), jnp.int32),
            pltpu.VMEM((n_grp, _GRP), jnp.int32),
            pltpu.VMEM((n_grp, _GRP), jnp.int32),
            pltpu.VMEM((_GRP, 128), jnp.float32),
            pltpu.VMEM((_GRP, 64), jnp.float32),
            pltpu.SemaphoreType.DMA((1,)),
        ],
    )
    return k(tab2, node_ids)


def kernel(node_ids, node_tids, type_ids, node_embed_weight):
    return _embed_lookup(node_ids.astype(jnp.int32), node_embed_weight)


# speedup vs baseline: 1.1137x; 1.1137x over previous
import functools

import jax
import jax.numpy as jnp
from jax import lax
from jax.experimental import pallas as pl
from jax.experimental.pallas import tpu as pltpu
from jax.experimental.pallas import tpu_sc as plsc

_GRP = 128
_L = 16


def _gather_body(
    b_per_w, tab2_hbm, ids_hbm, out_hbm, ids_v, q_v, hb_v, rows_v, stage_v, sems
):
    wid = lax.axis_index("s") * 2 + lax.axis_index("c")
    base = wid * b_per_w
    pltpu.sync_copy(ids_hbm.at[pl.ds(base, b_per_w)], ids_v)

    vec_per_grp = _GRP // _L
    for i in range(b_per_w // _L):
        v = ids_v[pl.ds(i * _L, _L)]
        g, o = i // vec_per_grp, (i % vec_per_grp) * _L
        q_v[g, pl.ds(o, _L)] = v
        hb_v[g, pl.ds(o, _L)] = v * 0

    for g in range(b_per_w // _GRP):
        pltpu.async_copy(tab2_hbm.at[q_v.at[g]], rows_v, sems.at[0]).wait()

        def per_batch(b, _):
            j0 = b * _L
            hv = hb_v[g, pl.ds(j0, _L)]
            for i in range(_L):
                j = j0 + i
                h = hv[i]
                for k in range(4):
                    stage_v[j, pl.ds(k * _L, _L)] = rows_v[
                        j, pl.ds(h + k * _L, _L)
                    ]
            return _

        lax.fori_loop(0, _GRP // _L, per_batch, 0, unroll=False)
        pltpu.sync_copy(
            stage_v, out_hbm.at[pl.ds(base + g * _GRP, _GRP)]
        )


@jax.jit
def _embed_lookup(node_ids, node_embed_weight):
    b = node_ids.shape[0]
    nrows = node_embed_weight.shape[0]
    tab2 = jnp.pad(node_embed_weight, ((0, 0), (0, 64)))
    info = plsc.get_sparse_core_info()
    nw = info.num_cores * info.num_subcores
    b_per_w = b // nw
    n_grp = b_per_w // _GRP
    mesh = plsc.VectorSubcoreMesh(core_axis_name="c", subcore_axis_name="s")
    k = pl.kernel(
        functools.partial(_gather_body, b_per_w),
        mesh=mesh,
        out_type=jax.ShapeDtypeStruct((b, 64), jnp.float32),
        scratch_types=[
            pltpu.VMEM((b_per_w,), jnp.int32),
            pltpu.VMEM((n_grp, _GRP), jnp.int32),
            pltpu.VMEM((n_grp, _GRP), jnp.int32),
            pltpu.VMEM((_GRP, 128), jnp.float32),
            pltpu.VMEM((_GRP, 64), jnp.float32),
            pltpu.SemaphoreType.DMA((1,)),
        ],
    )
    return k(tab2, node_ids)


def kernel(node_ids, node_tids, type_ids, node_embed_weight):
    return _embed_lookup(node_ids.astype(jnp.int32), node_embed_weight)


# speedup vs baseline: 1.6660x; 1.4959x over previous
import functools

import jax
import jax.numpy as jnp
from jax import lax
from jax.experimental import pallas as pl
from jax.experimental.pallas import tpu as pltpu
from jax.experimental.pallas import tpu_sc as plsc

_GRP = 128
_L = 16


def _gather_body(b_per_w, tab_hbm, ids_hbm, out_hbm, ids_v, stage_v, sems):
    wid = lax.axis_index("s") * 2 + lax.axis_index("c")
    base = wid * b_per_w
    pltpu.sync_copy(ids_hbm.at[pl.ds(base, b_per_w)], ids_v)

    for g in range(b_per_w // _GRP):

        def per_batch(b, _):
            vec = ids_v[pl.ds(g * _GRP + b * _L, _L)]
            for i in range(_L):
                r = vec[i]
                pltpu.make_async_copy(
                    tab_hbm.at[pl.ds(r, 1), :],
                    stage_v.at[pl.ds(b * _L + i, 1), :],
                    sems.at[0],
                ).start()
            for i in range(_L):
                pltpu.make_async_copy(
                    tab_hbm.at[pl.ds(0, 1), :],
                    stage_v.at[pl.ds(b * _L + i, 1), :],
                    sems.at[0],
                ).wait()
            return _

        lax.fori_loop(0, _GRP // _L, per_batch, 0, unroll=False)
        pltpu.sync_copy(stage_v, out_hbm.at[pl.ds(base + g * _GRP, _GRP)])


@jax.jit
def _embed_lookup(node_ids, node_embed_weight):
    b = node_ids.shape[0]
    info = plsc.get_sparse_core_info()
    nw = info.num_cores * info.num_subcores
    b_per_w = b // nw
    mesh = plsc.VectorSubcoreMesh(core_axis_name="c", subcore_axis_name="s")
    k = pl.kernel(
        functools.partial(_gather_body, b_per_w),
        mesh=mesh,
        out_type=jax.ShapeDtypeStruct((b, 64), jnp.float32),
        scratch_types=[
            pltpu.VMEM((b_per_w,), jnp.int32),
            pltpu.VMEM((_GRP, 64), jnp.float32),
            pltpu.SemaphoreType.DMA((1,)),
        ],
    )
    return k(node_embed_weight, node_ids)


def kernel(node_ids, node_tids, type_ids, node_embed_weight):
    return _embed_lookup(node_ids.astype(jnp.int32), node_embed_weight)
